# baseline (speedup 1.0000x reference)
import functools

import jax
import jax.numpy as jnp
from jax import lax
from jax.experimental import pallas as pl
from jax.experimental.pallas import tpu as pltpu
from jax.experimental.pallas import tpu_sc as plsc

NC = 2
NS = 16
NW = NC * NS
L = 16

B = 16
H = 512
W = 512
N_TOTAL = B * H * W

B_SC = 8
WPI = NW // B_SC
ROWS_PER_W = H // WPI
ROWS_PER_CHUNK = 32
N_CHUNKS = ROWS_PER_W // ROWS_PER_CHUNK
SLICES_PER_ROW = W // L

_mesh = plsc.VectorSubcoreMesh(core_axis_name="c", subcore_axis_name="s")


@functools.partial(
    pl.kernel,
    out_type=jax.ShapeDtypeStruct((NC, NS, 3, L), jnp.int32),
    mesh=_mesh,
    scratch_types=[
        pltpu.VMEM((2, ROWS_PER_CHUNK, W), jnp.float32),
        pltpu.VMEM((2, ROWS_PER_CHUNK, W), jnp.int32),
        pltpu.VMEM((3, L), jnp.int32),
        pltpu.VMEM((NS, 3, L), jnp.int32),
        pltpu.SemaphoreType.DMA,
        pltpu.SemaphoreType.DMA,
    ],
)
def _sc_kernel(pred_hbm, gt_hbm, out_hbm, pred_v, gt_v, part_v, buf_v,
               sem0, sem1):
    cid = lax.axis_index("c")
    sid = lax.axis_index("s")
    wid = sid * NC + cid
    b = wid // WPI
    r_base = (wid % WPI) * ROWS_PER_W
    sems = (sem0, sem1)

    def start(c):
        slot = c % 2
        r0 = r_base + c * ROWS_PER_CHUNK
        h1 = pltpu.async_copy(
            pred_hbm.at[b, pl.ds(r0, ROWS_PER_CHUNK), :], pred_v.at[slot],
            sems[slot])
        h2 = pltpu.async_copy(
            gt_hbm.at[b, pl.ds(r0, ROWS_PER_CHUNK), :], gt_v.at[slot],
            sems[slot])
        return (h1, h2)

    acc1 = jnp.zeros((L,), jnp.int32)
    acc2 = jnp.zeros((L,), jnp.int32)

    pending = start(0)
    for c in range(N_CHUNKS):
        slot = c % 2
        pending[0].wait()
        pending[1].wait()
        if c + 1 < N_CHUNKS:
            pending = start(c + 1)

        @plsc.parallel_loop(0, ROWS_PER_CHUNK, 1, unroll=2,
                            carry=(acc1, acc2))
        def body(i, accs):
            a1, a2 = accs
            for u in range(SLICES_PER_ROW):
                pv = pred_v[slot, i, pl.ds(u * L, L)]
                gv = gt_v[slot, i, pl.ds(u * L, L)]
                p = pv > 0.0
                a1 = a1 + jnp.where(p, gv + 65536, gv)
                a2 = a2 + jnp.where(p, gv, 0)
            return a1, a2

        acc1, acc2 = body

    part_v[0, :] = acc1 & 0xFFFF
    part_v[1, :] = lax.shift_right_logical(acc1, 16)
    part_v[2, :] = acc2

    pltpu.sync_copy(part_v, out_hbm.at[cid, sid])
    plsc.subcore_barrier()

    @pl.when(sid == 0)
    def _():
        pltpu.sync_copy(out_hbm.at[cid], buf_v)
        sg_v = jnp.zeros((L,), jnp.int32)
        sp_v = jnp.zeros((L,), jnp.int32)
        spg_v = jnp.zeros((L,), jnp.int32)
        for r in range(NS):
            sg_v = sg_v + buf_v[r, 0, :]
            sp_v = sp_v + buf_v[r, 1, :]
            spg_v = spg_v + buf_v[r, 2, :]
        part_v[0, :] = sg_v
        part_v[1, :] = sp_v
        part_v[2, :] = spg_v
        pltpu.sync_copy(part_v, out_hbm.at[cid, 0])


TC_ROWS = 128
TC_STEPS = (B - B_SC) * (H // TC_ROWS)


def _tc_body(pred_ref, gt_ref, out_ref, acc_ref):
    i = pl.program_id(0)

    @pl.when(i == 0)
    def _():
        acc_ref[0] = 0
        acc_ref[1] = 0
        acc_ref[2] = 0

    pb = pred_ref[0]
    gb = gt_ref[0]
    p = pb > 0.0
    one = jnp.ones_like(gb)
    zero = jnp.zeros_like(gb)
    acc_ref[0] += jnp.sum(gb)
    acc_ref[1] += jnp.sum(jnp.where(p, one, zero))
    acc_ref[2] += jnp.sum(jnp.where(p, gb, zero))

    @pl.when(i == TC_STEPS - 1)
    def _():
        out_ref[0] = acc_ref[0]
        out_ref[1] = acc_ref[1]
        out_ref[2] = acc_ref[2]


_tc_kernel = pl.pallas_call(
    _tc_body,
    grid=(TC_STEPS,),
    in_specs=[
        pl.BlockSpec((1, TC_ROWS, W),
                     lambda i: (B_SC + i // (H // TC_ROWS),
                                i % (H // TC_ROWS), 0)),
        pl.BlockSpec((1, TC_ROWS, W),
                     lambda i: (B_SC + i // (H // TC_ROWS),
                                i % (H // TC_ROWS), 0)),
    ],
    out_specs=pl.BlockSpec(memory_space=pltpu.SMEM),
    out_shape=jax.ShapeDtypeStruct((3,), jnp.int32),
    scratch_shapes=[pltpu.SMEM((3,), jnp.int32)],
)


def kernel(pred, gt):
    pred3 = pred.reshape(B, H, W)
    tc_parts = _tc_kernel(pred3, gt)
    sc_parts = _sc_kernel(pred3, gt)
    s = (sc_parts[0, 0] + sc_parts[1, 0]).sum(axis=1) + tc_parts
    sg, sp, spg = s[0], s[1], s[2]
    return jnp.array(
        [[N_TOTAL - sg - sp + spg, sp - spg], [sg - spg, spg]],
        dtype=jnp.int32)


# speedup vs baseline: 1.1673x; 1.1673x over previous
import functools

import jax
import jax.numpy as jnp
from jax import lax
from jax.experimental import pallas as pl
from jax.experimental.pallas import tpu as pltpu
from jax.experimental.pallas import tpu_sc as plsc

NC = 2
NS = 16
NW = NC * NS
L = 16

B = 16
H = 512
W = 512
N_TOTAL = B * H * W

B_SC = 8
WPI = NW // B_SC
ROWS_PER_W = H // WPI
ROWS_PER_CHUNK = 32
N_CHUNKS = ROWS_PER_W // ROWS_PER_CHUNK
SLICES_PER_ROW = W // L

_mesh = plsc.VectorSubcoreMesh(core_axis_name="c", subcore_axis_name="s")


@functools.partial(
    pl.kernel,
    out_type=jax.ShapeDtypeStruct((NC, NS, 3, L), jnp.int32),
    mesh=_mesh,
    scratch_types=[
        pltpu.VMEM((2, ROWS_PER_CHUNK, W), jnp.float32),
        pltpu.VMEM((2, ROWS_PER_CHUNK, W), jnp.int32),
        pltpu.VMEM((3, L), jnp.int32),
        pltpu.VMEM((NS, 3, L), jnp.int32),
        pltpu.SemaphoreType.DMA,
        pltpu.SemaphoreType.DMA,
    ],
)
def _sc_kernel(pred_hbm, gt_hbm, out_hbm, pred_v, gt_v, part_v, buf_v,
               sem0, sem1):
    cid = lax.axis_index("c")
    sid = lax.axis_index("s")
    wid = sid * NC + cid
    b = wid // WPI
    r_base = (wid % WPI) * ROWS_PER_W
    sems = (sem0, sem1)

    def start(c):
        slot = c % 2
        r0 = r_base + c * ROWS_PER_CHUNK
        h1 = pltpu.async_copy(
            pred_hbm.at[b, pl.ds(r0, ROWS_PER_CHUNK), :], pred_v.at[slot],
            sems[slot])
        h2 = pltpu.async_copy(
            gt_hbm.at[b, pl.ds(r0, ROWS_PER_CHUNK), :], gt_v.at[slot],
            sems[slot])
        return (h1, h2)

    acc1 = jnp.zeros((L,), jnp.int32)
    acc2 = jnp.zeros((L,), jnp.int32)

    pending = start(0)
    for c in range(N_CHUNKS):
        slot = c % 2
        pending[0].wait()
        pending[1].wait()
        if c + 1 < N_CHUNKS:
            pending = start(c + 1)

        @plsc.parallel_loop(0, ROWS_PER_CHUNK, 1, unroll=2,
                            carry=(acc1, acc2))
        def body(i, accs):
            a1, a2 = accs
            for u in range(SLICES_PER_ROW):
                pv = pred_v[slot, i, pl.ds(u * L, L)]
                gv = gt_v[slot, i, pl.ds(u * L, L)]
                p = pv > 0.0
                a1 = a1 + jnp.where(p, gv + 65536, gv)
                a2 = a2 + jnp.where(p, gv, 0)
            return a1, a2

        acc1, acc2 = body

    part_v[0, :] = acc1 & 0xFFFF
    part_v[1, :] = lax.shift_right_logical(acc1, 16)
    part_v[2, :] = acc2

    pltpu.sync_copy(part_v, out_hbm.at[cid, sid])
    plsc.subcore_barrier()

    @pl.when(sid == 0)
    def _():
        pltpu.sync_copy(out_hbm.at[cid], buf_v)
        sg_v = jnp.zeros((L,), jnp.int32)
        sp_v = jnp.zeros((L,), jnp.int32)
        spg_v = jnp.zeros((L,), jnp.int32)
        for r in range(NS):
            sg_v = sg_v + buf_v[r, 0, :]
            sp_v = sp_v + buf_v[r, 1, :]
            spg_v = spg_v + buf_v[r, 2, :]
        part_v[0, :] = sg_v
        part_v[1, :] = sp_v
        part_v[2, :] = spg_v
        pltpu.sync_copy(part_v, out_hbm.at[cid, 0])


TC_STEPS = B - B_SC


def _tc_body(pred_ref, gt_ref, out_ref, acc_ref):
    i = pl.program_id(0)

    @pl.when(i == 0)
    def _():
        acc_ref[...] = jnp.zeros_like(acc_ref)

    pb = pred_ref[0]
    gb = gt_ref[0]
    p = pb > 0.0
    gf = gb.astype(jnp.float32)
    pf = jnp.where(p, 1.0, 0.0)
    pgf = jnp.where(p, gf, 0.0)
    ones = jnp.ones((8, H), jnp.float32)
    acc_ref[0] += jax.lax.dot(ones, gf, preferred_element_type=jnp.float32)
    acc_ref[1] += jax.lax.dot(ones, pf, preferred_element_type=jnp.float32)
    acc_ref[2] += jax.lax.dot(ones, pgf, preferred_element_type=jnp.float32)

    @pl.when(i == TC_STEPS - 1)
    def _():
        out_ref[0] = jnp.sum(acc_ref[0, 0:1, :]).astype(jnp.int32)
        out_ref[1] = jnp.sum(acc_ref[1, 0:1, :]).astype(jnp.int32)
        out_ref[2] = jnp.sum(acc_ref[2, 0:1, :]).astype(jnp.int32)


_tc_kernel = pl.pallas_call(
    _tc_body,
    grid=(TC_STEPS,),
    in_specs=[
        pl.BlockSpec((1, H, W), lambda i: (B_SC + i, 0, 0)),
        pl.BlockSpec((1, H, W), lambda i: (B_SC + i, 0, 0)),
    ],
    out_specs=pl.BlockSpec(memory_space=pltpu.SMEM),
    out_shape=jax.ShapeDtypeStruct((3,), jnp.int32),
    scratch_shapes=[pltpu.VMEM((3, 8, W), jnp.float32)],
)


def kernel(pred, gt):
    pred3 = pred.reshape(B, H, W)
    tc_parts = _tc_kernel(pred3, gt)
    sc_parts = _sc_kernel(pred3, gt)
    s = (sc_parts[0, 0] + sc_parts[1, 0]).sum(axis=1) + tc_parts
    sg, sp, spg = s[0], s[1], s[2]
    return jnp.array(
        [[N_TOTAL - sg - sp + spg, sp - spg], [sg - spg, spg]],
        dtype=jnp.int32)
